# baseline (speedup 1.0000x reference)
import functools

import jax
import jax.numpy as jnp
from jax import lax
from jax.experimental import pallas as pl
from jax.experimental.pallas import tpu as pltpu
from jax.experimental.pallas import tpu_sc as plsc

_N = 4096
_M = 8192
_D = 512

_BI = 4096
_BJ = 1024
_JBLKS = _M // _BJ
_LANES = 128
_GRPS = _BJ // _LANES


def _argmin_body(x2_ref, y2_ref, xd_ref, y_ref, idx_ref, lv_ref, lc_ref):
    j = pl.program_id(1)

    mm = jax.lax.dot_general(
        2.0 * xd_ref[...], y_ref[...], (((1,), (1,)), ((), ())),
        preferred_element_type=jnp.float32)

    lane = lax.broadcasted_iota(
        jnp.int32, (_BI, _LANES), 1).astype(jnp.float32)
    x2 = x2_ref[...]

    @pl.when(j == 0)
    def _init():
        lv_ref[...] = jnp.full((_BI, _LANES), jnp.inf, jnp.float32)
        lc_ref[...] = jnp.zeros((_BI, _LANES), jnp.float32)

    lv = lv_ref[...]
    lc = lc_ref[...]
    for g in range(_GRPS):
        sqg = (x2 + y2_ref[:, g * _LANES:(g + 1) * _LANES]) \
            - mm[:, g * _LANES:(g + 1) * _LANES]
        col = lane + jnp.float32(j * _BJ + g * _LANES)
        upd = sqg < lv
        lc = jnp.where(upd, col, lc)
        lv = jnp.where(upd, sqg, lv)
    lv_ref[...] = lv
    lc_ref[...] = lc

    @pl.when(j == _JBLKS - 1)
    def _fin():
        d = jnp.sqrt(jnp.maximum(lv, 0.0))
        m = jnp.min(d, axis=1, keepdims=True)
        cand = jnp.min(jnp.where(d == m, lc, jnp.float32(_M)),
                       axis=1, keepdims=True)
        idx_ref[...] = cand.astype(jnp.int32)


def _nearest_idx(x, y, x2, y2):
    n_loc = x.shape[0]
    return pl.pallas_call(
        _argmin_body,
        grid=(n_loc // _BI, _JBLKS),
        in_specs=[
            pl.BlockSpec((_BI, 1), lambda i, j: (i, 0)),
            pl.BlockSpec((1, _BJ), lambda i, j: (0, j)),
            pl.BlockSpec((_BI, _D), lambda i, j: (i, 0)),
            pl.BlockSpec((_BJ, _D), lambda i, j: (j, 0)),
        ],
        out_specs=pl.BlockSpec((_BI, 1), lambda i, j: (i, 0)),
        out_shape=jax.ShapeDtypeStruct((n_loc, 1), jnp.int32),
        scratch_shapes=[
            pltpu.VMEM((_BI, _LANES), jnp.float32),
            pltpu.VMEM((_BI, _LANES), jnp.float32),
        ],
        compiler_params=pltpu.CompilerParams(
            dimension_semantics=("parallel", "arbitrary")),
    )(x2, y2, x, y)


def _make_sc_gather(n_loc):
    info = plsc.get_sparse_core_info()
    nc, ns = info.num_cores, info.num_subcores
    nw = nc * ns
    b_per_w = n_loc // nw
    mesh = plsc.VectorSubcoreMesh(core_axis_name="c", subcore_axis_name="s")

    @functools.partial(
        pl.kernel, mesh=mesh,
        out_type=jax.ShapeDtypeStruct((n_loc, _D), jnp.float32),
        scratch_types=[
            pltpu.VMEM((b_per_w,), jnp.int32),
            pltpu.VMEM((b_per_w, _D), jnp.float32),
            pltpu.SemaphoreType.DMA,
        ],
    )
    def _gather(table_hbm, idx_hbm, out_hbm, idx_v, rows_v, sem):
        wid = lax.axis_index("s") * nc + lax.axis_index("c")
        base = wid * b_per_w
        pltpu.sync_copy(idx_hbm.at[pl.ds(base, b_per_w)], idx_v)
        pltpu.async_copy(table_hbm.at[idx_v], rows_v, sem).wait()
        pltpu.sync_copy(rows_v, out_hbm.at[pl.ds(base, b_per_w)])

    return _gather


_BF = 1024


def _fuse_body(x_ref, a_ref, w1a_ref, w1b_ref, b1_ref, w2_ref, b2_ref, o_ref):
    x = x_ref[...]
    a = a_ref[...]
    h = (jnp.dot(x, w1a_ref[...], preferred_element_type=jnp.float32)
         + jnp.dot(a, w1b_ref[...], preferred_element_type=jnp.float32)
         + b1_ref[...])
    h = jnp.maximum(h, 0.0)
    z = jnp.sum(h * w2_ref[...], axis=1, keepdims=True) + b2_ref[...]
    w = jax.nn.sigmoid(z)
    o_ref[...] = w * x + (1.0 - w) * a


def _fuse(x, aligned, w1a, w1b, b1, w2row, b2):
    n_loc = x.shape[0]
    return pl.pallas_call(
        _fuse_body,
        grid=(n_loc // _BF,),
        in_specs=[
            pl.BlockSpec((_BF, _D), lambda i: (i, 0)),
            pl.BlockSpec((_BF, _D), lambda i: (i, 0)),
            pl.BlockSpec((_D, _D), lambda i: (0, 0)),
            pl.BlockSpec((_D, _D), lambda i: (0, 0)),
            pl.BlockSpec((1, _D), lambda i: (0, 0)),
            pl.BlockSpec((1, _D), lambda i: (0, 0)),
            pl.BlockSpec((1, 1), lambda i: (0, 0)),
        ],
        out_specs=pl.BlockSpec((_BF, _D), lambda i: (i, 0)),
        out_shape=jax.ShapeDtypeStruct((n_loc, _D), jnp.float32),
        compiler_params=pltpu.CompilerParams(
            dimension_semantics=("parallel",)),
    )(x, aligned, w1a, w1b, b1, w2row, b2)


def kernel(clear_feature, rain_feature, W1, b1, W2, b2):
    x, y = clear_feature, rain_feature
    x2 = jnp.sum(x * x, axis=-1, keepdims=True)
    y2 = jnp.sum(y * y, axis=-1, keepdims=True).T

    idx = _nearest_idx(x, y, x2, y2)
    aligned = _make_sc_gather(_N)(y, idx.reshape(_N))

    w1a = W1[:_D]
    w1b = W1[_D:]
    return _fuse(x, aligned, w1a, w1b, b1.reshape(1, _D),
                 W2.reshape(1, _D), b2.reshape(1, 1))


# speedup vs baseline: 1.0148x; 1.0148x over previous
import functools

import jax
import jax.numpy as jnp
from jax import lax
from jax.experimental import pallas as pl
from jax.experimental.pallas import tpu as pltpu
from jax.experimental.pallas import tpu_sc as plsc

_N = 4096
_M = 8192
_D = 512

_BI = 2048
_BJ = 2048
_JBLKS = _M // _BJ
_LANES = 128
_GRPS = _BJ // _LANES


def _argmin_body(x2_ref, y2_ref, xd_ref, y_ref, idx_ref, lv_ref, lc_ref):
    j = pl.program_id(1)

    mm = jax.lax.dot_general(
        2.0 * xd_ref[...], y_ref[...], (((1,), (1,)), ((), ())),
        preferred_element_type=jnp.float32)

    lane = lax.broadcasted_iota(
        jnp.int32, (_BI, _LANES), 1).astype(jnp.float32)
    x2 = x2_ref[...]

    @pl.when(j == 0)
    def _init():
        lv_ref[...] = jnp.full((_BI, _LANES), jnp.inf, jnp.float32)
        lc_ref[...] = jnp.zeros((_BI, _LANES), jnp.float32)

    lv = lv_ref[...]
    lc = lc_ref[...]
    for g in range(_GRPS):
        sqg = (x2 + y2_ref[:, g * _LANES:(g + 1) * _LANES]) \
            - mm[:, g * _LANES:(g + 1) * _LANES]
        col = lane + jnp.float32(j * _BJ + g * _LANES)
        upd = sqg < lv
        lc = jnp.where(upd, col, lc)
        lv = jnp.where(upd, sqg, lv)
    lv_ref[...] = lv
    lc_ref[...] = lc

    @pl.when(j == _JBLKS - 1)
    def _fin():
        d = jnp.sqrt(jnp.maximum(lv, 0.0))
        m = jnp.min(d, axis=1, keepdims=True)
        cand = jnp.min(jnp.where(d == m, lc, jnp.float32(_M)),
                       axis=1, keepdims=True)
        idx_ref[...] = cand.astype(jnp.int32)


def _nearest_idx(x, y, x2, y2):
    n_loc = x.shape[0]
    return pl.pallas_call(
        _argmin_body,
        grid=(n_loc // _BI, _JBLKS),
        in_specs=[
            pl.BlockSpec((_BI, 1), lambda i, j: (i, 0)),
            pl.BlockSpec((1, _BJ), lambda i, j: (0, j)),
            pl.BlockSpec((_BI, _D), lambda i, j: (i, 0)),
            pl.BlockSpec((_BJ, _D), lambda i, j: (j, 0)),
        ],
        out_specs=pl.BlockSpec((_BI, 1), lambda i, j: (i, 0)),
        out_shape=jax.ShapeDtypeStruct((n_loc, 1), jnp.int32),
        scratch_shapes=[
            pltpu.VMEM((_BI, _LANES), jnp.float32),
            pltpu.VMEM((_BI, _LANES), jnp.float32),
        ],
        compiler_params=pltpu.CompilerParams(
            dimension_semantics=("parallel", "arbitrary")),
    )(x2, y2, x, y)


def _make_sc_gather(n_loc):
    info = plsc.get_sparse_core_info()
    nc, ns = info.num_cores, info.num_subcores
    nw = nc * ns
    b_per_w = n_loc // nw
    mesh = plsc.VectorSubcoreMesh(core_axis_name="c", subcore_axis_name="s")

    @functools.partial(
        pl.kernel, mesh=mesh,
        out_type=jax.ShapeDtypeStruct((n_loc, _D), jnp.float32),
        scratch_types=[
            pltpu.VMEM((b_per_w,), jnp.int32),
            pltpu.VMEM((b_per_w, _D), jnp.float32),
            pltpu.SemaphoreType.DMA,
        ],
    )
    def _gather(table_hbm, idx_hbm, out_hbm, idx_v, rows_v, sem):
        wid = lax.axis_index("s") * nc + lax.axis_index("c")
        base = wid * b_per_w
        pltpu.sync_copy(idx_hbm.at[pl.ds(base, b_per_w)], idx_v)
        pltpu.async_copy(table_hbm.at[idx_v], rows_v, sem).wait()
        pltpu.sync_copy(rows_v, out_hbm.at[pl.ds(base, b_per_w)])

    return _gather


_BF = 1024


def _fuse_body(x_ref, a_ref, w1a_ref, w1b_ref, b1_ref, w2_ref, b2_ref, o_ref):
    x = x_ref[...]
    a = a_ref[...]
    h = (jnp.dot(x, w1a_ref[...], preferred_element_type=jnp.float32)
         + jnp.dot(a, w1b_ref[...], preferred_element_type=jnp.float32)
         + b1_ref[...])
    h = jnp.maximum(h, 0.0)
    z = jnp.sum(h * w2_ref[...], axis=1, keepdims=True) + b2_ref[...]
    w = jax.nn.sigmoid(z)
    o_ref[...] = w * x + (1.0 - w) * a


def _fuse(x, aligned, w1a, w1b, b1, w2row, b2):
    n_loc = x.shape[0]
    return pl.pallas_call(
        _fuse_body,
        grid=(n_loc // _BF,),
        in_specs=[
            pl.BlockSpec((_BF, _D), lambda i: (i, 0)),
            pl.BlockSpec((_BF, _D), lambda i: (i, 0)),
            pl.BlockSpec((_D, _D), lambda i: (0, 0)),
            pl.BlockSpec((_D, _D), lambda i: (0, 0)),
            pl.BlockSpec((1, _D), lambda i: (0, 0)),
            pl.BlockSpec((1, _D), lambda i: (0, 0)),
            pl.BlockSpec((1, 1), lambda i: (0, 0)),
        ],
        out_specs=pl.BlockSpec((_BF, _D), lambda i: (i, 0)),
        out_shape=jax.ShapeDtypeStruct((n_loc, _D), jnp.float32),
        compiler_params=pltpu.CompilerParams(
            dimension_semantics=("parallel",)),
    )(x, aligned, w1a, w1b, b1, w2row, b2)


def kernel(clear_feature, rain_feature, W1, b1, W2, b2):
    x, y = clear_feature, rain_feature
    x2 = jnp.sum(x * x, axis=-1, keepdims=True)
    y2 = jnp.sum(y * y, axis=-1, keepdims=True).T

    idx = _nearest_idx(x, y, x2, y2)
    aligned = _make_sc_gather(_N)(y, idx.reshape(_N))

    w1a = W1[:_D]
    w1b = W1[_D:]
    return _fuse(x, aligned, w1a, w1b, b1.reshape(1, _D),
                 W2.reshape(1, _D), b2.reshape(1, 1))
